# baseline (speedup 1.0000x reference)
import functools

import jax
import jax.numpy as jnp
from jax import lax
from jax.experimental import pallas as pl
from jax.experimental.layout import Layout, with_layout_constraint
from jax.experimental.pallas import tpu as pltpu
from jax.experimental.pallas import tpu_sc as plsc


@functools.lru_cache(maxsize=None)
def _make_gather_mean(B, L, V, D):
  info = plsc.get_sparse_core_info()
  NC, NS, NL = info.num_cores, info.num_subcores, info.num_lanes
  NW = NC * NS
  assert B % NW == 0
  nb = B // NW
  C1 = 128
  C2 = L - C1
  assert 0 < C2 <= 128 and C2 % 8 == 0 and L % 8 == 0 and D % NL == 0
  nv = D // NL
  inv_l = 1.0 / L
  LP = ((L + NL - 1) // NL) * NL
  W = 2 * D

  mesh = plsc.VectorSubcoreMesh(core_axis_name="c", subcore_axis_name="s")

  @functools.partial(
      pl.kernel,
      mesh=mesh,
      compiler_params=pltpu.CompilerParams(use_tc_tiling_on_sc=True),
      out_type=jax.ShapeDtypeStruct((B, D), jnp.float32),
      scratch_types=[
          pltpu.VMEM((nb, L), jnp.int32),
          pltpu.VMEM((2, LP), jnp.int32),
          pltpu.VMEM((2, L, W), jnp.float32),
          pltpu.VMEM((nb, D), jnp.float32),
          pltpu.SemaphoreType.DMA,
          pltpu.SemaphoreType.DMA,
      ],
  )
  def gather_mean(x_hbm, tab2_hbm, out_hbm, idx_v, idx2_v, rows_v, acc_v,
                  sem0, sem1):
    wid = lax.axis_index("s") * NC + lax.axis_index("c")
    base = wid * nb
    sems = (sem0, sem1)

    pltpu.sync_copy(x_hbm.at[pl.ds(base, nb), :], idx_v)

    def shift_idx(j, b):
      offs = [t * NL for t in range(L // NL)]
      if L % NL:
        offs.append(L - NL)
      for o in offs:
        idx2_v[b, pl.ds(o, NL)] = lax.shift_right_logical(
            idx_v[j, pl.ds(o, NL)], 1)

    def copies(j, b):
      del j
      return (
          pltpu.make_async_copy(tab2_hbm.at[idx2_v.at[b].at[pl.ds(0, C1)]],
                                rows_v.at[b].at[pl.ds(0, C1), :], sems[b]),
          pltpu.make_async_copy(tab2_hbm.at[idx2_v.at[b].at[pl.ds(C1, C2)]],
                                rows_v.at[b].at[pl.ds(C1, C2), :], sems[b]),
      )

    def issue(j, b):
      shift_idx(j, b)
      for c in copies(j, b):
        c.start()

    def drain(j, b):
      for c in copies(j, b):
        c.wait()

    def accum(j, b):
      def step(vo, rr, accs):
        vidx = idx_v[j, pl.ds(vo, NL)]
        half = pl.multiple_of((vidx[rr] & 1) * D, 8)
        r = vo + rr
        return tuple(accs[k] + rows_v[b, r, pl.ds(half + k * NL, NL)]
                     for k in range(nv))

      def body(it, accs):
        vo = it * NL
        for rr in range(NL):
          accs = step(vo, rr, accs)
        return accs

      zeros = tuple(jnp.zeros((NL,), jnp.float32) for _ in range(nv))
      accs = lax.fori_loop(0, L // NL, body, zeros)
      for rr in range(NL - (L - NL * (L // NL)), NL):
        accs = step(L - NL, rr, accs)
      for k in range(nv):
        acc_v[j, pl.ds(k * NL, NL)] = accs[k] * inv_l

    issue(0, 0)
    issue(1, 1)

    def outer(i2, carry):
      for b in range(2):
        j = i2 * 2 + b
        drain(j, b)
        accum(j, b)

        @pl.when(j + 2 < nb)
        def _():
          issue(j + 2, b)
      return carry

    lax.fori_loop(0, nb // 2, outer, 0)

    pltpu.sync_copy(acc_v, out_hbm.at[pl.ds(base, nb), :])

  return gather_mean


def _mlp(h, W1, b1, W2, b2, Wo, bo):
  B, D = h.shape
  BB = 1024
  dn = (((1,), (1,)), ((), ()))

  def body(h_ref, w1_ref, b1_ref, w2_ref, b2_ref, wo_ref, bo_ref, out_ref):
    hh = h_ref[...]
    h1 = jnp.maximum(
        lax.dot_general(hh, w1_ref[...], dn,
                        preferred_element_type=jnp.float32) + b1_ref[...], 0.0)
    h2 = jnp.maximum(
        lax.dot_general(h1, w2_ref[...], dn,
                        preferred_element_type=jnp.float32) + b2_ref[...], 0.0)
    out_ref[...] = lax.dot_general(
        h2, wo_ref[...], dn,
        preferred_element_type=jnp.float32) + bo_ref[...]

  return pl.pallas_call(
      body,
      grid=(B // BB,),
      in_specs=[
          pl.BlockSpec((BB, D), lambda i: (i, 0)),
          pl.BlockSpec(W1.shape, lambda i: (0, 0)),
          pl.BlockSpec((1, b1.shape[0]), lambda i: (0, 0)),
          pl.BlockSpec(W2.shape, lambda i: (0, 0)),
          pl.BlockSpec((1, b2.shape[0]), lambda i: (0, 0)),
          pl.BlockSpec(Wo.shape, lambda i: (0, 0)),
          pl.BlockSpec((1, bo.shape[0]), lambda i: (0, 0)),
      ],
      out_specs=pl.BlockSpec((BB, Wo.shape[0]), lambda i: (i, 0)),
      out_shape=jax.ShapeDtypeStruct((B, Wo.shape[0]), jnp.float32),
  )(h, W1, b1.reshape(1, -1), W2, b2.reshape(1, -1), Wo, bo.reshape(1, -1))


def kernel(x, table, W1, b1, W2, b2, Wo, bo):
  B, L = x.shape
  V, D = table.shape
  table_rm = with_layout_constraint(
      table, Layout(major_to_minor=(0, 1), tiling=((8, D),)))
  tab2 = with_layout_constraint(
      table_rm.reshape(V // 2, 2 * D),
      Layout(major_to_minor=(0, 1), tiling=((8, 2 * D),)))
  h = _make_gather_mean(B, L, V, D)(x, tab2)
  return _mlp(h, W1, b1, W2, b2, Wo, bo)


# speedup vs baseline: 1.6812x; 1.6812x over previous
import functools

import jax
import jax.numpy as jnp
from jax import lax
from jax.experimental import pallas as pl
from jax.experimental.layout import Layout, with_layout_constraint
from jax.experimental.pallas import tpu as pltpu
from jax.experimental.pallas import tpu_sc as plsc


@functools.lru_cache(maxsize=None)
def _make_gather_mean(B, L, V, D):
  info = plsc.get_sparse_core_info()
  NC, NS, NL = info.num_cores, info.num_subcores, info.num_lanes
  NW = NC * NS
  assert B % NW == 0
  nb = B // NW
  C1 = 128
  C2 = L - C1
  assert 0 < C2 <= 128 and C2 % 8 == 0 and L % 8 == 0 and D % NL == 0
  nv = D // NL
  inv_l = 1.0 / L

  mesh = plsc.VectorSubcoreMesh(core_axis_name="c", subcore_axis_name="s")

  @functools.partial(
      pl.kernel,
      mesh=mesh,
      compiler_params=pltpu.CompilerParams(use_tc_tiling_on_sc=False),
      out_type=jax.ShapeDtypeStruct((B, D), jnp.float32),
      scratch_types=[
          pltpu.VMEM((nb, L), jnp.int32),
          pltpu.VMEM((2, L, D), jnp.float32),
          pltpu.VMEM((nb, D), jnp.float32),
          pltpu.SemaphoreType.DMA,
          pltpu.SemaphoreType.DMA,
      ],
  )
  def gather_mean(x_hbm, table_hbm, out_hbm, idx_v, rows_v, acc_v, sem0, sem1):
    wid = lax.axis_index("s") * NC + lax.axis_index("c")
    base = wid * nb
    sems = (sem0, sem1)

    pltpu.sync_copy(x_hbm.at[pl.ds(base, nb), :], idx_v)

    def copies(j, b):
      return (
          pltpu.make_async_copy(table_hbm.at[idx_v.at[j, pl.ds(0, C1)]],
                                rows_v.at[b].at[pl.ds(0, C1), :], sems[b]),
          pltpu.make_async_copy(table_hbm.at[idx_v.at[j, pl.ds(C1, C2)]],
                                rows_v.at[b].at[pl.ds(C1, C2), :], sems[b]),
      )

    def issue(j, b):
      for c in copies(j, b):
        c.start()

    def drain(j, b):
      for c in copies(j, b):
        c.wait()

    def accum(j, b):
      def body(it, accs):
        r0 = it * 8
        for rr in range(8):
          r = r0 + rr
          accs = tuple(accs[k] + rows_v[b, r, pl.ds(k * NL, NL)]
                       for k in range(nv))
        return accs
      zeros = tuple(jnp.zeros((NL,), jnp.float32) for _ in range(nv))
      accs = lax.fori_loop(0, L // 8, body, zeros)
      for k in range(nv):
        acc_v[j, pl.ds(k * NL, NL)] = accs[k] * inv_l

    issue(0, 0)
    issue(1, 1)

    def outer(i2, carry):
      for b in range(2):
        j = i2 * 2 + b
        drain(j, b)
        accum(j, b)

        @pl.when(j + 2 < nb)
        def _():
          issue(j + 2, b)
      return carry

    lax.fori_loop(0, nb // 2, outer, 0)

    pltpu.sync_copy(acc_v, out_hbm.at[pl.ds(base, nb), :])

  return gather_mean


def _mlp(h, W1, b1, W2, b2, Wo, bo):
  B, D = h.shape
  BB = 1024
  dn = (((1,), (1,)), ((), ()))

  def body(h_ref, w1_ref, b1_ref, w2_ref, b2_ref, wo_ref, bo_ref, out_ref):
    hh = h_ref[...]
    h1 = jnp.maximum(
        lax.dot_general(hh, w1_ref[...], dn,
                        preferred_element_type=jnp.float32) + b1_ref[...], 0.0)
    h2 = jnp.maximum(
        lax.dot_general(h1, w2_ref[...], dn,
                        preferred_element_type=jnp.float32) + b2_ref[...], 0.0)
    out_ref[...] = lax.dot_general(
        h2, wo_ref[...], dn,
        preferred_element_type=jnp.float32) + bo_ref[...]

  return pl.pallas_call(
      body,
      grid=(B // BB,),
      in_specs=[
          pl.BlockSpec((BB, D), lambda i: (i, 0)),
          pl.BlockSpec(W1.shape, lambda i: (0, 0)),
          pl.BlockSpec((1, b1.shape[0]), lambda i: (0, 0)),
          pl.BlockSpec(W2.shape, lambda i: (0, 0)),
          pl.BlockSpec((1, b2.shape[0]), lambda i: (0, 0)),
          pl.BlockSpec(Wo.shape, lambda i: (0, 0)),
          pl.BlockSpec((1, bo.shape[0]), lambda i: (0, 0)),
      ],
      out_specs=pl.BlockSpec((BB, Wo.shape[0]), lambda i: (i, 0)),
      out_shape=jax.ShapeDtypeStruct((B, Wo.shape[0]), jnp.float32),
  )(h, W1, b1.reshape(1, -1), W2, b2.reshape(1, -1), Wo, bo.reshape(1, -1))


def kernel(x, table, W1, b1, W2, b2, Wo, bo):
  B, L = x.shape
  V, D = table.shape
  table_rm = with_layout_constraint(
      table, Layout(major_to_minor=(0, 1), tiling=((8, 128),)))
  h = _make_gather_mean(B, L, V, D)(x, table_rm)
  return _mlp(h, W1, b1, W2, b2, Wo, bo)
